# baseline (device time: 152610 ns/iter reference)
import jax
import jax.numpy as jnp
from jax import lax
from jax.experimental import pallas as pl
from jax.experimental.pallas import tpu as pltpu

T_HALF = 1024
T = 2 * T_HALF
D = 1024
F = 4096
F_TILE = 1024
E_LOC = 8
N_E = 2 * E_LOC
CAP = 320


def _exchange_kernel(x_shard, router_shard):

    def body(x_ref, r_ref, xo_ref, slot_ref, w_ref, gates, rfull,
             sx, rx, sr, rr, sg, rg):
        my_x = lax.axis_index("x")
        my_y = lax.axis_index("y")
        my_z = lax.axis_index("z")
        nbr = (1 - my_x, my_y, my_z)

        barrier_sem = pltpu.get_barrier_semaphore()
        pl.semaphore_signal(barrier_sem, inc=1, device_id=nbr,
                            device_id_type=pl.DeviceIdType.MESH)
        pl.semaphore_wait(barrier_sem, 1)


        rfull[pl.ds(my_x, 1)] = r_ref[...][None]
        rdma_r = pltpu.make_async_remote_copy(
            src_ref=r_ref, dst_ref=rfull.at[my_x],
            send_sem=sr, recv_sem=rr,
            device_id=nbr, device_id_type=pl.DeviceIdType.MESH,
        )
        rdma_r.start()
        rdma_r.wait()

        router_full = jnp.concatenate([rfull[0], rfull[1]], axis=1)
        g_mine = jnp.dot(x_ref[...], router_full,
                         preferred_element_type=jnp.float32,
                         precision=lax.Precision.HIGHEST)
        gates[pl.ds(my_x * T_HALF, T_HALF), :] = g_mine
        rdma_g = pltpu.make_async_remote_copy(
            src_ref=gates.at[pl.ds(my_x * T_HALF, T_HALF)],
            dst_ref=gates.at[pl.ds(my_x * T_HALF, T_HALF)],
            send_sem=sg, recv_sem=rg,
            device_id=nbr, device_id_type=pl.DeviceIdType.MESH,
        )
        rdma_g.start()
        rdma_g.wait()

        xo_ref[pl.ds(my_x * T_HALF, T_HALF), :] = \
            x_ref[...].astype(jnp.bfloat16)
        rdma_x = pltpu.make_async_remote_copy(
            src_ref=xo_ref.at[pl.ds(my_x * T_HALF, T_HALF)],
            dst_ref=xo_ref.at[pl.ds(my_x * T_HALF, T_HALF)],
            send_sem=sx, recv_sem=rx,
            device_id=nbr, device_id_type=pl.DeviceIdType.MESH,
        )
        rdma_x.start()

        g_t = jnp.transpose(gates[...])
        ids = lax.broadcasted_iota(jnp.int32, (N_E, T), 0)
        m1 = jnp.max(g_t, axis=0, keepdims=True)
        top1 = jnp.argmax(g_t, axis=0).reshape(1, T)
        masked = jnp.where(ids == top1, -jnp.inf, g_t)
        m2 = jnp.max(masked, axis=0, keepdims=True)
        top2 = jnp.argmax(masked, axis=0).reshape(1, T)
        w1g = 1.0 / (1.0 + jnp.exp(m2 - m1))
        w2g = 1.0 - w1g

        eids = lax.broadcasted_iota(jnp.int32, (E_LOC, T), 0) + my_x * E_LOC
        sel1 = top1 == eids
        sel2 = top2 == eids
        ind = sel1 | sel2
        w_ref[...] = jnp.where(sel1, w1g, 0.0) + jnp.where(sel2, w2g, 0.0)

        pos = ind.astype(jnp.int32)
        k = 1
        while k < T:
            shifted = jnp.concatenate(
                [jnp.zeros((E_LOC, k), jnp.int32), pos[:, :-k]], axis=1)
            pos = pos + shifted
            k *= 2
        slot_ref[...] = jnp.where(ind, pos - 1, -1)

        rdma_x.wait()

    return pl.pallas_call(
        body,
        out_shape=[
            jax.ShapeDtypeStruct((T, D), jnp.bfloat16),
            jax.ShapeDtypeStruct((E_LOC, T), jnp.int32),
            jax.ShapeDtypeStruct((E_LOC, T), jnp.float32),
        ],
        in_specs=[
            pl.BlockSpec(memory_space=pltpu.VMEM),
            pl.BlockSpec(memory_space=pltpu.VMEM),
        ],
        out_specs=[
            pl.BlockSpec(memory_space=pltpu.VMEM),
            pl.BlockSpec(memory_space=pltpu.VMEM),
            pl.BlockSpec(memory_space=pltpu.VMEM),
        ],
        scratch_shapes=[
            pltpu.VMEM((T, N_E), jnp.float32),
            pltpu.VMEM((2, D, E_LOC), jnp.float32),
            pltpu.SemaphoreType.DMA,
            pltpu.SemaphoreType.DMA,
            pltpu.SemaphoreType.DMA,
            pltpu.SemaphoreType.DMA,
            pltpu.SemaphoreType.DMA,
            pltpu.SemaphoreType.DMA,
        ],
        compiler_params=pltpu.CompilerParams(collective_id=0),
    )(x_shard, router_shard)


def _moe_kernel(x_all, slot_t, w_t, W1, W2, q):

    def body(q_ref, x_ref, slot_ref, w_ref, w1_ref, w2_ref, partial_ref,
             d_mat, xg):
        e = pl.program_id(0)

        cap_ids = lax.broadcasted_iota(jnp.int32, (CAP, T), 0)
        d_bool = cap_ids == slot_ref[pl.ds(e, 1)]
        d_mat[...] = d_bool.astype(jnp.bfloat16)
        xg[...] = jnp.dot(d_mat[...], x_ref[...],
                          preferred_element_type=jnp.float32
                          ).astype(jnp.bfloat16)

        h = jnp.dot(xg[...], w1_ref[0].astype(jnp.bfloat16),
                    preferred_element_type=jnp.float32)
        h = jnp.maximum(h, 0.0).astype(jnp.bfloat16)
        contrib = jnp.dot(h, w2_ref[0].astype(jnp.bfloat16),
                          preferred_element_type=jnp.float32)

        w_row = w_ref[pl.ds(e, 1)].astype(jnp.bfloat16)
        d_mat[...] = d_mat[...] * w_row
        s = lax.dot_general(d_mat[...], contrib.astype(jnp.bfloat16),
                            dimension_numbers=(((0,), (0,)), ((), ())),
                            preferred_element_type=jnp.float32)

        @pl.when(e == 0)
        def _():
            partial_ref[...] = s.astype(jnp.bfloat16)

        @pl.when(e != 0)
        def _():
            partial_ref[...] = (
                partial_ref[...].astype(jnp.float32) + s
            ).astype(jnp.bfloat16)

    grid_spec = pltpu.PrefetchScalarGridSpec(
        num_scalar_prefetch=1,
        grid=(E_LOC,),
        in_specs=[
            pl.BlockSpec((T, D), lambda e, q: (0, 0)),
            pl.BlockSpec((E_LOC, T), lambda e, q: (0, 0)),
            pl.BlockSpec((E_LOC, T), lambda e, q: (0, 0)),
            pl.BlockSpec((1, D, F_TILE), lambda e, q: (e, 0, q[0])),
            pl.BlockSpec((1, F_TILE, D), lambda e, q: (e, q[0], 0)),
        ],
        out_specs=pl.BlockSpec((T, D), lambda e, q: (0, 0)),
        scratch_shapes=[
            pltpu.VMEM((CAP, T), jnp.bfloat16),
            pltpu.VMEM((CAP, D), jnp.bfloat16),
        ],
    )
    return pl.pallas_call(
        body,
        grid_spec=grid_spec,
        out_shape=jax.ShapeDtypeStruct((T, D), jnp.bfloat16),
        compiler_params=pltpu.CompilerParams(
            dimension_semantics=("arbitrary",),
        ),
    )(q, x_all, slot_t, w_t, W1, W2)


def _combine_kernel(partial):

    n_ch = 8
    ch = T_HALF // n_ch

    def body(p_ref, out_ref, acc, sendb_y, sendb_z, comm_x, comm_y, comm_z,
             sx, rx, sy, ry, sz, rz):
        my_x = lax.axis_index("x")
        my_y = lax.axis_index("y")
        my_z = lax.axis_index("z")
        nbr_x = (1 - my_x, my_y, my_z)
        nbr_y = (my_x, 1 - my_y, my_z)
        nbr_z = (my_x, my_y, 1 - my_z)

        barrier_sem = pltpu.get_barrier_semaphore()
        for nbr in (nbr_x, nbr_y, nbr_z):
            pl.semaphore_signal(barrier_sem, inc=1, device_id=nbr,
                                device_id_type=pl.DeviceIdType.MESH)
        pl.semaphore_wait(barrier_sem, 3)

        def cs(c):
            return pl.ds(c * ch, ch)

        rd_x, rd_y, rd_z = [], [], []
        for c in range(n_ch):
            rd = pltpu.make_async_remote_copy(
                src_ref=p_ref.at[pl.ds((1 - my_x) * T_HALF + c * ch, ch)],
                dst_ref=comm_x.at[cs(c)], send_sem=sx.at[c], recv_sem=rx.at[c],
                device_id=nbr_x, device_id_type=pl.DeviceIdType.MESH,
            )
            rd.start()
            rd_x.append(rd)
        for c in range(n_ch):
            rd_x[c].wait()
            mine = p_ref[pl.ds(my_x * T_HALF + c * ch, ch), :]
            acc[cs(c), :] = (mine.astype(jnp.float32)
                             + comm_x[cs(c), :].astype(jnp.float32))
            sendb_y[cs(c), :] = acc[cs(c), :].astype(jnp.bfloat16)
            rd = pltpu.make_async_remote_copy(
                src_ref=sendb_y.at[cs(c)], dst_ref=comm_y.at[cs(c)],
                send_sem=sy.at[c], recv_sem=ry.at[c],
                device_id=nbr_y, device_id_type=pl.DeviceIdType.MESH,
            )
            rd.start()
            rd_y.append(rd)
        for c in range(n_ch):
            rd_y[c].wait()
            acc[cs(c), :] += comm_y[cs(c), :].astype(jnp.float32)
            sendb_z[cs(c), :] = acc[cs(c), :].astype(jnp.bfloat16)
            rd = pltpu.make_async_remote_copy(
                src_ref=sendb_z.at[cs(c)], dst_ref=comm_z.at[cs(c)],
                send_sem=sz.at[c], recv_sem=rz.at[c],
                device_id=nbr_z, device_id_type=pl.DeviceIdType.MESH,
            )
            rd.start()
            rd_z.append(rd)
        for c in range(n_ch):
            rd_z[c].wait()
            out_ref[cs(c), :] = acc[cs(c), :] + comm_z[cs(c), :].astype(
                jnp.float32)

    return pl.pallas_call(
        body,
        out_shape=jax.ShapeDtypeStruct((T_HALF, D), jnp.float32),
        in_specs=[pl.BlockSpec(memory_space=pltpu.VMEM)],
        out_specs=pl.BlockSpec(memory_space=pltpu.VMEM),
        scratch_shapes=[
            pltpu.VMEM((T_HALF, D), jnp.float32),
            pltpu.VMEM((T_HALF, D), jnp.bfloat16),
            pltpu.VMEM((T_HALF, D), jnp.bfloat16),
            pltpu.VMEM((T_HALF, D), jnp.bfloat16),
            pltpu.VMEM((T_HALF, D), jnp.bfloat16),
            pltpu.VMEM((T_HALF, D), jnp.bfloat16),
            pltpu.SemaphoreType.DMA((n_ch,)),
            pltpu.SemaphoreType.DMA((n_ch,)),
            pltpu.SemaphoreType.DMA((n_ch,)),
            pltpu.SemaphoreType.DMA((n_ch,)),
            pltpu.SemaphoreType.DMA((n_ch,)),
            pltpu.SemaphoreType.DMA((n_ch,)),
        ],
        compiler_params=pltpu.CompilerParams(collective_id=1),
    )(partial)


def kernel(x, router, W1, W2):
    my_y = lax.axis_index("y")
    my_z = lax.axis_index("z")
    q = jnp.reshape(my_y * 2 + my_z, (1,)).astype(jnp.int32)

    x_all, slot_t, w_t = _exchange_kernel(x, router)
    partial = _moe_kernel(x_all, slot_t, w_t, W1, W2, q)
    return _combine_kernel(partial)


# device time: 132326 ns/iter; 1.1533x vs baseline; 1.1533x over previous
import jax
import jax.numpy as jnp
from jax import lax
from jax.experimental import pallas as pl
from jax.experimental.pallas import tpu as pltpu

T_HALF = 1024
T = 2 * T_HALF
D = 1024
F = 4096
F_TILE = 1024
N_F = F // F_TILE
E_LOC = 8
N_E = 2 * E_LOC
CAP = 320


def _exchange_kernel(x_shard, router_shard):

    def body(x_ref, r_ref, xo_ref, slot_ref, w_ref, gates, rfull,
             sx, rx, sr, rr, sg, rg):
        my_x = lax.axis_index("x")
        my_y = lax.axis_index("y")
        my_z = lax.axis_index("z")
        nbr = (1 - my_x, my_y, my_z)

        barrier_sem = pltpu.get_barrier_semaphore()
        pl.semaphore_signal(barrier_sem, inc=1, device_id=nbr,
                            device_id_type=pl.DeviceIdType.MESH)
        pl.semaphore_wait(barrier_sem, 1)


        rfull[pl.ds(my_x, 1)] = r_ref[...][None]
        rdma_r = pltpu.make_async_remote_copy(
            src_ref=r_ref, dst_ref=rfull.at[my_x],
            send_sem=sr, recv_sem=rr,
            device_id=nbr, device_id_type=pl.DeviceIdType.MESH,
        )
        rdma_r.start()
        rdma_r.wait()

        router_full = jnp.concatenate([rfull[0], rfull[1]], axis=1)
        g_mine = jnp.dot(x_ref[...], router_full,
                         preferred_element_type=jnp.float32,
                         precision=lax.Precision.HIGHEST)
        gates[pl.ds(my_x * T_HALF, T_HALF), :] = g_mine
        rdma_g = pltpu.make_async_remote_copy(
            src_ref=gates.at[pl.ds(my_x * T_HALF, T_HALF)],
            dst_ref=gates.at[pl.ds(my_x * T_HALF, T_HALF)],
            send_sem=sg, recv_sem=rg,
            device_id=nbr, device_id_type=pl.DeviceIdType.MESH,
        )
        rdma_g.start()
        rdma_g.wait()

        xo_ref[pl.ds(my_x * T_HALF, T_HALF), :] = \
            x_ref[...].astype(jnp.bfloat16)
        rdma_x = pltpu.make_async_remote_copy(
            src_ref=xo_ref.at[pl.ds(my_x * T_HALF, T_HALF)],
            dst_ref=xo_ref.at[pl.ds(my_x * T_HALF, T_HALF)],
            send_sem=sx, recv_sem=rx,
            device_id=nbr, device_id_type=pl.DeviceIdType.MESH,
        )
        rdma_x.start()

        g_t = jnp.transpose(gates[...])
        ids = lax.broadcasted_iota(jnp.int32, (N_E, T), 0)
        m1 = jnp.max(g_t, axis=0, keepdims=True)
        top1 = jnp.argmax(g_t, axis=0).reshape(1, T)
        masked = jnp.where(ids == top1, -jnp.inf, g_t)
        m2 = jnp.max(masked, axis=0, keepdims=True)
        top2 = jnp.argmax(masked, axis=0).reshape(1, T)
        w1g = 1.0 / (1.0 + jnp.exp(m2 - m1))
        w2g = 1.0 - w1g

        eids = lax.broadcasted_iota(jnp.int32, (E_LOC, T), 0) + my_x * E_LOC
        sel1 = top1 == eids
        sel2 = top2 == eids
        ind = sel1 | sel2
        w_ref[...] = jnp.where(sel1, w1g, 0.0) + jnp.where(sel2, w2g, 0.0)

        pos = ind.astype(jnp.int32)
        k = 1
        while k < T:
            shifted = jnp.concatenate(
                [jnp.zeros((E_LOC, k), jnp.int32), pos[:, :-k]], axis=1)
            pos = pos + shifted
            k *= 2
        slot_ref[...] = jnp.where(ind, pos - 1, -1)

        rdma_x.wait()

    return pl.pallas_call(
        body,
        out_shape=[
            jax.ShapeDtypeStruct((T, D), jnp.bfloat16),
            jax.ShapeDtypeStruct((E_LOC, T), jnp.int32),
            jax.ShapeDtypeStruct((E_LOC, T), jnp.float32),
        ],
        in_specs=[
            pl.BlockSpec(memory_space=pltpu.VMEM),
            pl.BlockSpec(memory_space=pltpu.VMEM),
        ],
        out_specs=[
            pl.BlockSpec(memory_space=pltpu.VMEM),
            pl.BlockSpec(memory_space=pltpu.VMEM),
            pl.BlockSpec(memory_space=pltpu.VMEM),
        ],
        scratch_shapes=[
            pltpu.VMEM((T, N_E), jnp.float32),
            pltpu.VMEM((2, D, E_LOC), jnp.float32),
            pltpu.SemaphoreType.DMA,
            pltpu.SemaphoreType.DMA,
            pltpu.SemaphoreType.DMA,
            pltpu.SemaphoreType.DMA,
            pltpu.SemaphoreType.DMA,
            pltpu.SemaphoreType.DMA,
        ],
        compiler_params=pltpu.CompilerParams(collective_id=0),
    )(x_shard, router_shard)


def _moe_kernel(x_all, slot_t, w_t, W1, W2, q):

    def body(q_ref, x_ref, slot_ref, w_ref, w1_ref, w2_ref, partial_ref,
             d_mat, xg, y_acc):
        e = pl.program_id(0)
        f = pl.program_id(1)
        el = q_ref[0] * 2 + e

        @pl.when(f == 0)
        def _gather():
            cap_ids = lax.broadcasted_iota(jnp.int32, (CAP, T), 0)
            d_bool = cap_ids == slot_ref[pl.ds(el, 1)]
            d_mat[...] = d_bool.astype(jnp.bfloat16)
            xg[...] = jnp.dot(d_mat[...], x_ref[...],
                              preferred_element_type=jnp.float32
                              ).astype(jnp.bfloat16)

        h = jnp.dot(xg[...], w1_ref[0].astype(jnp.bfloat16),
                    preferred_element_type=jnp.float32)
        h = jnp.maximum(h, 0.0).astype(jnp.bfloat16)
        contrib = jnp.dot(h, w2_ref[0].astype(jnp.bfloat16),
                          preferred_element_type=jnp.float32)

        @pl.when(f == 0)
        def _():
            y_acc[...] = contrib

        @pl.when(f != 0)
        def _():
            y_acc[...] += contrib

        @pl.when(f == N_F - 1)
        def _scatter():
            w_row = w_ref[pl.ds(el, 1)].astype(jnp.bfloat16)
            d_mat[...] = d_mat[...] * w_row
            s = lax.dot_general(d_mat[...], y_acc[...].astype(jnp.bfloat16),
                                dimension_numbers=(((0,), (0,)), ((), ())),
                                preferred_element_type=jnp.float32)

            @pl.when(e == 0)
            def _():
                partial_ref[...] = s.astype(jnp.bfloat16)

            @pl.when(e != 0)
            def _():
                partial_ref[...] = (
                    partial_ref[...].astype(jnp.float32) + s
                ).astype(jnp.bfloat16)

    grid_spec = pltpu.PrefetchScalarGridSpec(
        num_scalar_prefetch=1,
        grid=(2, N_F),
        in_specs=[
            pl.BlockSpec((T, D), lambda e, f, q: (0, 0)),
            pl.BlockSpec((E_LOC, T), lambda e, f, q: (0, 0)),
            pl.BlockSpec((E_LOC, T), lambda e, f, q: (0, 0)),
            pl.BlockSpec((1, D, F_TILE), lambda e, f, q: (q[0] * 2 + e, 0, f)),
            pl.BlockSpec((1, F_TILE, D), lambda e, f, q: (q[0] * 2 + e, f, 0)),
        ],
        out_specs=pl.BlockSpec((T, D), lambda e, f, q: (0, 0)),
        scratch_shapes=[
            pltpu.VMEM((CAP, T), jnp.bfloat16),
            pltpu.VMEM((CAP, D), jnp.bfloat16),
            pltpu.VMEM((CAP, D), jnp.float32),
        ],
    )
    return pl.pallas_call(
        body,
        grid_spec=grid_spec,
        out_shape=jax.ShapeDtypeStruct((T, D), jnp.bfloat16),
        compiler_params=pltpu.CompilerParams(
            dimension_semantics=("arbitrary", "arbitrary"),
        ),
    )(q, x_all, slot_t, w_t, W1, W2)


def _combine_kernel(partial):

    n_ch = 8
    ch = T_HALF // n_ch

    def body(p_ref, out_ref, acc, sendb_y, sendb_z, comm_x, comm_y, comm_z,
             sx, rx, sy, ry, sz, rz):
        my_x = lax.axis_index("x")
        my_y = lax.axis_index("y")
        my_z = lax.axis_index("z")
        nbr_x = (1 - my_x, my_y, my_z)
        nbr_y = (my_x, 1 - my_y, my_z)
        nbr_z = (my_x, my_y, 1 - my_z)

        barrier_sem = pltpu.get_barrier_semaphore()
        for nbr in (nbr_x, nbr_y, nbr_z):
            pl.semaphore_signal(barrier_sem, inc=1, device_id=nbr,
                                device_id_type=pl.DeviceIdType.MESH)
        pl.semaphore_wait(barrier_sem, 3)

        def cs(c):
            return pl.ds(c * ch, ch)

        rd_x, rd_y, rd_z = [], [], []
        for c in range(n_ch):
            rd = pltpu.make_async_remote_copy(
                src_ref=p_ref.at[pl.ds((1 - my_x) * T_HALF + c * ch, ch)],
                dst_ref=comm_x.at[cs(c)], send_sem=sx.at[c], recv_sem=rx.at[c],
                device_id=nbr_x, device_id_type=pl.DeviceIdType.MESH,
            )
            rd.start()
            rd_x.append(rd)
        for c in range(n_ch):
            rd_x[c].wait()
            mine = p_ref[pl.ds(my_x * T_HALF + c * ch, ch), :]
            acc[cs(c), :] = (mine.astype(jnp.float32)
                             + comm_x[cs(c), :].astype(jnp.float32))
            sendb_y[cs(c), :] = acc[cs(c), :].astype(jnp.bfloat16)
            rd = pltpu.make_async_remote_copy(
                src_ref=sendb_y.at[cs(c)], dst_ref=comm_y.at[cs(c)],
                send_sem=sy.at[c], recv_sem=ry.at[c],
                device_id=nbr_y, device_id_type=pl.DeviceIdType.MESH,
            )
            rd.start()
            rd_y.append(rd)
        for c in range(n_ch):
            rd_y[c].wait()
            acc[cs(c), :] += comm_y[cs(c), :].astype(jnp.float32)
            sendb_z[cs(c), :] = acc[cs(c), :].astype(jnp.bfloat16)
            rd = pltpu.make_async_remote_copy(
                src_ref=sendb_z.at[cs(c)], dst_ref=comm_z.at[cs(c)],
                send_sem=sz.at[c], recv_sem=rz.at[c],
                device_id=nbr_z, device_id_type=pl.DeviceIdType.MESH,
            )
            rd.start()
            rd_z.append(rd)
        for c in range(n_ch):
            rd_z[c].wait()
            out_ref[cs(c), :] = acc[cs(c), :] + comm_z[cs(c), :].astype(
                jnp.float32)

    return pl.pallas_call(
        body,
        out_shape=jax.ShapeDtypeStruct((T_HALF, D), jnp.float32),
        in_specs=[pl.BlockSpec(memory_space=pltpu.VMEM)],
        out_specs=pl.BlockSpec(memory_space=pltpu.VMEM),
        scratch_shapes=[
            pltpu.VMEM((T_HALF, D), jnp.float32),
            pltpu.VMEM((T_HALF, D), jnp.bfloat16),
            pltpu.VMEM((T_HALF, D), jnp.bfloat16),
            pltpu.VMEM((T_HALF, D), jnp.bfloat16),
            pltpu.VMEM((T_HALF, D), jnp.bfloat16),
            pltpu.VMEM((T_HALF, D), jnp.bfloat16),
            pltpu.SemaphoreType.DMA((n_ch,)),
            pltpu.SemaphoreType.DMA((n_ch,)),
            pltpu.SemaphoreType.DMA((n_ch,)),
            pltpu.SemaphoreType.DMA((n_ch,)),
            pltpu.SemaphoreType.DMA((n_ch,)),
            pltpu.SemaphoreType.DMA((n_ch,)),
        ],
        compiler_params=pltpu.CompilerParams(collective_id=1),
    )(partial)


def kernel(x, router, W1, W2):
    my_y = lax.axis_index("y")
    my_z = lax.axis_index("z")
    q = jnp.reshape(my_y * 2 + my_z, (1,)).astype(jnp.int32)

    x_all, slot_t, w_t = _exchange_kernel(x, router)
    partial = _moe_kernel(x_all, slot_t, w_t, W1, W2, q)
    return _combine_kernel(partial)
